# Initial kernel scaffold; baseline (speedup 1.0000x reference)
import jax
import jax.numpy as jnp
from jax.experimental import pallas as pl


def kernel(feature_1, feature_2, table_1, table_2):
    raise NotImplementedError("write your pallas kernel here")


# speedup vs baseline: 310.6519x; 310.6519x over previous
import jax
import jax.numpy as jnp
from jax.experimental import pallas as pl

B, L = 16384, 200
VOCAB, DIM = 10, 3
ROWS = 512


def _body(f1_ref, f2_ref, t1_ref, t2_ref, o_ref):
    f1 = f1_ref[...]
    f2 = f2_ref[...]
    h1 = jnp.stack(
        [jnp.sum((f1 == v).astype(jnp.float32), axis=1) for v in range(VOCAB)],
        axis=1,
    )
    h2 = jnp.stack(
        [jnp.sum((f2 == v).astype(jnp.float32), axis=1) for v in range(VOCAB)],
        axis=1,
    )
    acc = jnp.dot(h1, t1_ref[...], preferred_element_type=jnp.float32)
    acc += jnp.dot(h2, t2_ref[...], preferred_element_type=jnp.float32)
    o_ref[...] = acc * jnp.float32(1.0 / L)


def kernel(feature_1, feature_2, table_1, table_2):
    grid = (B // ROWS,)
    return pl.pallas_call(
        _body,
        grid=grid,
        in_specs=[
            pl.BlockSpec((ROWS, L), lambda i: (i, 0)),
            pl.BlockSpec((ROWS, L), lambda i: (i, 0)),
            pl.BlockSpec((VOCAB, DIM), lambda i: (0, 0)),
            pl.BlockSpec((VOCAB, DIM), lambda i: (0, 0)),
        ],
        out_specs=pl.BlockSpec((ROWS, DIM), lambda i: (i, 0)),
        out_shape=jax.ShapeDtypeStruct((B, DIM), jnp.float32),
    )(feature_1, feature_2, table_1, table_2)
